# baseline (speedup 1.0000x reference)
import jax
import jax.numpy as jnp
from jax import lax
from jax.experimental import pallas as pl
from jax.experimental.pallas import tpu as pltpu

EPS = 1e-07
N = 2048
DIM = 768
H = 12
HD = DIM // H
K = 16
PDIM = 16
ROWS = 256
GRID = N // ROWS
NEG = -1e30
INF = 3.0e38
LOG2E = 1.4426950408889634
_HI = lax.Precision.HIGHEST


def _qkv_body(x_ref, s_ref, b_ref, wqkv_ref, bqkv_ref, q_ref, k_ref, v_ref):
    x = x_ref[...]
    m = jnp.mean(x, axis=1, keepdims=True)
    xc = x - m
    var = jnp.mean(xc * xc, axis=1, keepdims=True)
    xn = (xc / jnp.sqrt(var + 1e-6) * s_ref[...] + b_ref[...]).astype(jnp.bfloat16)
    qkv = (jnp.dot(xn, wqkv_ref[...], preferred_element_type=jnp.float32)
           + bqkv_ref[...]).astype(jnp.bfloat16)
    q_ref[...] = qkv[:, :DIM]
    k_ref[...] = qkv[:, DIM:2 * DIM]
    v_ref[...] = qkv[:, 2 * DIM:]


def _attn_body(q_ref, x_ref, pos_ref, k_ref, v_ref, wt_ref, posbd_ref,
               ascr_ref, c_ref, fsc_ref, wout_ref, bout_ref, s2_ref, b2_ref,
               wf1_ref, bf1_ref, wf2_ref, bf2_ref, out_ref):
    i = pl.program_id(0)
    c = c_ref[0, 0]
    sqrt_c = jnp.maximum(jnp.sqrt(c), EPS)
    max_norm = (1.0 - EPS) / sqrt_c

    pos = pos_ref[...]
    pos_blk = pos_ref[pl.ds(i * ROWS, ROWS), :]
    q = q_ref[...]

    si = jnp.sum(pos_blk * pos_blk, axis=1, keepdims=True)
    ones_p = jnp.ones((1, PDIM), jnp.float32)
    sj = lax.dot_general(ones_p, pos * pos, (((1,), (1,)), ((), ())),
                         precision=_HI, preferred_element_type=jnp.float32)
    g = lax.dot_general(pos_blk, pos, (((1,), (1,)), ((), ())),
                        precision=_HI, preferred_element_type=jnp.float32)

    two_cg = 2.0 * c * g
    d0 = 1.0 - two_cg + (c * c) * si * sj
    den = jnp.maximum(d0, EPS)

    a1 = 1.0 - two_cg + c * si
    b1 = 1.0 - c * sj
    r2 = (a1 * a1 * sj - 2.0 * a1 * b1 * g + b1 * b1 * si) / (den * den)

    iota = lax.broadcasted_iota(jnp.int32, (ROWS, N), 1)

    work = jnp.minimum(r2, INF * 0.5)
    for _ in range(K):
        mrow = jnp.min(work, axis=1, keepdims=True)
        candidx = jnp.where(work == mrow, iota, N)
        jm = jnp.min(candidx, axis=1, keepdims=True)
        sel = candidx == jm
        work = jnp.where(sel, INF, work)
    nbias = jnp.where(work == INF, 0.0, NEG)

    a2 = 1.0 - two_cg + c * sj
    b2 = 1.0 - c * si
    nrm2 = jnp.maximum(a2 * a2 * si - 2.0 * a2 * b2 * g + b2 * b2 * sj, 0.0)
    nrm_u = jnp.sqrt(nrm2) / den
    scale = jnp.minimum(max_norm / jnp.maximum(nrm_u, EPS), 1.0)
    norm_m = nrm_u * scale
    safe = jnp.maximum(norm_m, EPS)
    arg = jnp.minimum(sqrt_c * safe, 1.0 - EPS)
    mag = 0.5 * jnp.log((1.0 + arg) / (1.0 - arg)) / sqrt_c
    gam = jnp.where(norm_m < EPS, 0.0, mag / safe)
    sden = scale / den
    p_co = gam * (-a2 * sden)
    q_co = gam * (b2 * sden)

    qg_all = jnp.dot(q.astype(jnp.bfloat16), wt_ref[...],
                     preferred_element_type=jnp.float32) * ascr_ref[...]
    g2all = jnp.dot(qg_all.astype(jnp.bfloat16), posbd_ref[...],
                    preferred_element_type=jnp.float32).astype(jnp.bfloat16)

    inv_sqrt_hd = 1.0 / jnp.sqrt(jnp.float32(HD))
    outs = []
    for h in range(H):
        sl = slice(h * HD, (h + 1) * HD)
        q_h = q[:, sl]
        qf = q_h * (fsc_ref[0, h] * (inv_sqrt_hd * LOG2E)).astype(jnp.bfloat16)
        feat = lax.dot_general(qf, k_ref[:, sl], (((1,), (1,)), ((), ())),
                               preferred_element_type=jnp.float32)
        qg = qg_all[:, h * PDIM:(h + 1) * PDIM]
        g1 = jnp.sum(qg * pos_blk, axis=1, keepdims=True)
        g2 = g2all[:, h * N:(h + 1) * N].astype(jnp.float32)
        s = feat + p_co * g1 + q_co * g2 + nbias
        mx = jnp.max(s, axis=1, keepdims=True)
        e = jnp.exp2(s - mx)
        rden = 1.0 / jnp.sum(e, axis=1, keepdims=True)
        ov = jnp.dot(e.astype(jnp.bfloat16), v_ref[:, sl],
                     preferred_element_type=jnp.float32)
        outs.append(ov * rden)
    attn = jnp.concatenate(outs, axis=1).astype(jnp.bfloat16)

    xo = x_ref[...] + jnp.dot(attn, wout_ref[...],
                              preferred_element_type=jnp.float32) + bout_ref[...]
    m = jnp.mean(xo, axis=1, keepdims=True)
    xc = xo - m
    var = jnp.mean(xc * xc, axis=1, keepdims=True)
    hn = (xc / jnp.sqrt(var + 1e-6) * s2_ref[...] + b2_ref[...]).astype(jnp.bfloat16)
    h1 = jax.nn.gelu(jnp.dot(hn, wf1_ref[...],
                             preferred_element_type=jnp.float32) + bf1_ref[...])
    out_ref[...] = xo + jnp.dot(h1.astype(jnp.bfloat16), wf2_ref[...],
                                preferred_element_type=jnp.float32) + bf2_ref[...]


_PAR = pltpu.CompilerParams(dimension_semantics=("parallel",))


def _blk(r, d):
    return pl.BlockSpec((r, d), lambda i: (i, 0))


def _full(shape):
    return pl.BlockSpec(shape, lambda i: (0, 0))


def kernel(x_item, positions_item, c_item, Wq, bq, Wk, bk, Wv, bv, Wgeo, bgeo,
           Wout, bout, Wf1, bf1, Wf2, bf2, ln1_s, ln1_b, ln2_s, ln2_b,
           alignment_scale, feature_scale):
    pos = positions_item.astype(jnp.float32)
    row = lambda a: a.reshape(1, -1)
    c2 = c_item.reshape(1, 1)
    f32 = jnp.float32

    Wqkv = jnp.concatenate([Wq, Wk, Wv], axis=1).astype(jnp.bfloat16)
    bqkv = jnp.concatenate([bq, bk, bv])[None, :]
    q, k, v = pl.pallas_call(
        _qkv_body,
        grid=(GRID,),
        in_specs=[_blk(ROWS, DIM), _full((1, DIM)), _full((1, DIM)),
                  _full((DIM, 3 * DIM)), _full((1, 3 * DIM))],
        out_specs=[_blk(ROWS, DIM)] * 3,
        out_shape=[jax.ShapeDtypeStruct((N, DIM), jnp.bfloat16)] * 3,
        compiler_params=_PAR,
    )(x_item, row(ln1_s), row(ln1_b), Wqkv, bqkv)

    bf16 = jnp.bfloat16
    Wt = jnp.zeros((DIM, H * PDIM), f32)
    PosBD = jnp.zeros((H * PDIM, H * N), bf16)
    posT = pos.T.astype(bf16)
    for h in range(H):
        Wt = Wt.at[h * HD:(h + 1) * HD, h * PDIM:(h + 1) * PDIM].set(
            Wgeo[:, h * HD:(h + 1) * HD].T)
        PosBD = PosBD.at[h * PDIM:(h + 1) * PDIM, h * N:(h + 1) * N].set(posT)
    ascr = (jnp.repeat(alignment_scale, PDIM) * LOG2E)[None, :]

    out = pl.pallas_call(
        _attn_body,
        grid=(GRID,),
        in_specs=[_blk(ROWS, DIM), _blk(ROWS, DIM), _full((N, PDIM)),
                  _full((N, DIM)), _full((N, DIM)), _full((DIM, H * PDIM)),
                  _full((H * PDIM, H * N)), _full((1, H * PDIM)),
                  _full((1, 1)), _full((1, H)),
                  _full((DIM, DIM)), _full((1, DIM)),
                  _full((1, DIM)), _full((1, DIM)),
                  _full((DIM, 4 * DIM)), _full((1, 4 * DIM)),
                  _full((4 * DIM, DIM)), _full((1, DIM))],
        out_specs=_blk(ROWS, DIM),
        out_shape=jax.ShapeDtypeStruct((N, DIM), f32),
        compiler_params=_PAR,
    )(q, x_item, pos, k, v, Wt.astype(bf16), PosBD, ascr, c2,
      row(feature_scale), Wout.astype(bf16), row(bout), row(ln2_s),
      row(ln2_b), Wf1.astype(bf16), row(bf1), Wf2.astype(bf16), row(bf2))
    return out


# speedup vs baseline: 1.1940x; 1.1940x over previous
import jax
import jax.numpy as jnp
from jax import lax
from jax.experimental import pallas as pl
from jax.experimental.pallas import tpu as pltpu

EPS = 1e-07
N = 2048
DIM = 768
H = 12
HD = DIM // H
K = 16
PDIM = 16
ROWS = 256
GRID = N // ROWS
NEG = -1e30
INF = 3.0e38
LOG2E = 1.4426950408889634
_HI = lax.Precision.HIGHEST


def _qkv_body(x_ref, s_ref, b_ref, wqkv_ref, bqkv_ref, q_ref, k_ref, v_ref):
    x = x_ref[...]
    m = jnp.mean(x, axis=1, keepdims=True)
    xc = x - m
    var = jnp.mean(xc * xc, axis=1, keepdims=True)
    xn = (xc / jnp.sqrt(var + 1e-6) * s_ref[...] + b_ref[...]).astype(jnp.bfloat16)
    qkv = (jnp.dot(xn, wqkv_ref[...], preferred_element_type=jnp.float32)
           + bqkv_ref[...]).astype(jnp.bfloat16)
    q_ref[...] = qkv[:, :DIM]
    k_ref[...] = qkv[:, DIM:2 * DIM]
    v_ref[...] = qkv[:, 2 * DIM:]


def _attn_body(q_ref, pos_ref, k_ref, v_ref, wt_ref, posbd_ref, ascr_ref,
               c_ref, fsc_ref, out_ref):
    i = pl.program_id(0)
    c = c_ref[0, 0]
    sqrt_c = jnp.maximum(jnp.sqrt(c), EPS)
    max_norm = (1.0 - EPS) / sqrt_c

    pos = pos_ref[...]
    pos_blk = pos_ref[pl.ds(i * ROWS, ROWS), :]
    q = q_ref[...]

    si = jnp.sum(pos_blk * pos_blk, axis=1, keepdims=True)
    ones_p = jnp.ones((1, PDIM), jnp.float32)
    sj = lax.dot_general(ones_p, pos * pos, (((1,), (1,)), ((), ())),
                         precision=_HI, preferred_element_type=jnp.float32)
    g = lax.dot_general(pos_blk, pos, (((1,), (1,)), ((), ())),
                        precision=_HI, preferred_element_type=jnp.float32)

    two_cg = 2.0 * c * g
    d0 = 1.0 - two_cg + (c * c) * si * sj
    den = jnp.maximum(d0, EPS)

    a1 = 1.0 - two_cg + c * si
    b1 = 1.0 - c * sj
    r2 = (a1 * a1 * sj - 2.0 * a1 * b1 * g + b1 * b1 * si) / (den * den)

    iota = lax.broadcasted_iota(jnp.int32, (ROWS, N), 1)

    work = jnp.minimum(r2, INF * 0.5)
    for _ in range(K):
        mrow = jnp.min(work, axis=1, keepdims=True)
        candidx = jnp.where(work == mrow, iota, N)
        jm = jnp.min(candidx, axis=1, keepdims=True)
        sel = candidx == jm
        work = jnp.where(sel, INF, work)
    nbias = jnp.where(work == INF, 0.0, NEG)

    a2 = 1.0 - two_cg + c * sj
    b2 = 1.0 - c * si
    nrm2 = jnp.maximum(a2 * a2 * si - 2.0 * a2 * b2 * g + b2 * b2 * sj, 0.0)
    nrm_u = jnp.sqrt(nrm2) / den
    scale = jnp.minimum(max_norm / jnp.maximum(nrm_u, EPS), 1.0)
    norm_m = nrm_u * scale
    safe = jnp.maximum(norm_m, EPS)
    arg = jnp.minimum(sqrt_c * safe, 1.0 - EPS)
    mag = 0.5 * jnp.log((1.0 + arg) / (1.0 - arg)) / sqrt_c
    gam = jnp.where(norm_m < EPS, 0.0, mag / safe)
    sden = scale / den
    p_co = gam * (-a2 * sden)
    q_co = gam * (b2 * sden)

    qg_all = jnp.dot(q.astype(jnp.bfloat16), wt_ref[...],
                     preferred_element_type=jnp.float32) * ascr_ref[...]
    g2all = jnp.dot(qg_all.astype(jnp.bfloat16), posbd_ref[...],
                    preferred_element_type=jnp.float32).astype(jnp.bfloat16)

    inv_sqrt_hd = 1.0 / jnp.sqrt(jnp.float32(HD))
    for h in range(H):
        sl = slice(h * HD, (h + 1) * HD)
        q_h = q[:, sl]
        qf = q_h * (fsc_ref[0, h] * (inv_sqrt_hd * LOG2E)).astype(jnp.bfloat16)
        feat = lax.dot_general(qf, k_ref[:, sl], (((1,), (1,)), ((), ())),
                               preferred_element_type=jnp.float32)
        qg = qg_all[:, h * PDIM:(h + 1) * PDIM]
        g1 = jnp.sum(qg * pos_blk, axis=1, keepdims=True)
        g2 = g2all[:, h * N:(h + 1) * N].astype(jnp.float32)
        s = feat + p_co * g1 + q_co * g2 + nbias
        mx = jnp.max(s, axis=1, keepdims=True)
        e = jnp.exp2(s - mx)
        rden = 1.0 / jnp.sum(e, axis=1, keepdims=True)
        ov = jnp.dot(e.astype(jnp.bfloat16), v_ref[:, sl],
                     preferred_element_type=jnp.float32)
        out_ref[:, sl] = ov * rden


def _ffn_body(x_ref, attn_ref, wout_ref, bout_ref, s2_ref, b2_ref, wf1_ref,
              bf1_ref, wf2_ref, bf2_ref, o_ref):
    xo = x_ref[...] + jnp.dot(attn_ref[...].astype(jnp.bfloat16), wout_ref[...],
                              preferred_element_type=jnp.float32) + bout_ref[...]
    m = jnp.mean(xo, axis=1, keepdims=True)
    xc = xo - m
    var = jnp.mean(xc * xc, axis=1, keepdims=True)
    hn = (xc / jnp.sqrt(var + 1e-6) * s2_ref[...] + b2_ref[...]).astype(jnp.bfloat16)
    h1 = jax.nn.gelu(jnp.dot(hn, wf1_ref[...],
                             preferred_element_type=jnp.float32) + bf1_ref[...])
    o_ref[...] = xo + jnp.dot(h1.astype(jnp.bfloat16), wf2_ref[...],
                              preferred_element_type=jnp.float32) + bf2_ref[...]


_PAR = pltpu.CompilerParams(dimension_semantics=("parallel",))


def _blk(r, d):
    return pl.BlockSpec((r, d), lambda i: (i, 0))


def _full(shape):
    return pl.BlockSpec(shape, lambda i: (0, 0))


def kernel(x_item, positions_item, c_item, Wq, bq, Wk, bk, Wv, bv, Wgeo, bgeo,
           Wout, bout, Wf1, bf1, Wf2, bf2, ln1_s, ln1_b, ln2_s, ln2_b,
           alignment_scale, feature_scale):
    pos = positions_item.astype(jnp.float32)
    row = lambda a: a.reshape(1, -1)
    c2 = c_item.reshape(1, 1)
    f32 = jnp.float32

    Wqkv = jnp.concatenate([Wq, Wk, Wv], axis=1).astype(jnp.bfloat16)
    bqkv = jnp.concatenate([bq, bk, bv])[None, :]
    q, k, v = pl.pallas_call(
        _qkv_body,
        grid=(GRID,),
        in_specs=[_blk(ROWS, DIM), _full((1, DIM)), _full((1, DIM)),
                  _full((DIM, 3 * DIM)), _full((1, 3 * DIM))],
        out_specs=[_blk(ROWS, DIM)] * 3,
        out_shape=[jax.ShapeDtypeStruct((N, DIM), jnp.bfloat16)] * 3,
        compiler_params=_PAR,
    )(x_item, row(ln1_s), row(ln1_b), Wqkv, bqkv)

    bf16 = jnp.bfloat16
    Wt = jnp.zeros((DIM, H * PDIM), f32)
    PosBD = jnp.zeros((H * PDIM, H * N), bf16)
    posT = pos.T.astype(bf16)
    for h in range(H):
        Wt = Wt.at[h * HD:(h + 1) * HD, h * PDIM:(h + 1) * PDIM].set(
            Wgeo[:, h * HD:(h + 1) * HD].T)
        PosBD = PosBD.at[h * PDIM:(h + 1) * PDIM, h * N:(h + 1) * N].set(posT)
    ascr = (jnp.repeat(alignment_scale, PDIM) * LOG2E)[None, :]

    attn = pl.pallas_call(
        _attn_body,
        grid=(GRID,),
        in_specs=[_blk(ROWS, DIM), _full((N, PDIM)), _full((N, DIM)),
                  _full((N, DIM)), _full((DIM, H * PDIM)),
                  _full((H * PDIM, H * N)), _full((1, H * PDIM)),
                  _full((1, 1)), _full((1, H))],
        out_specs=_blk(ROWS, DIM),
        out_shape=jax.ShapeDtypeStruct((N, DIM), f32),
        compiler_params=_PAR,
    )(q, pos, k, v, Wt.astype(bf16), PosBD, ascr, c2, row(feature_scale))

    out = pl.pallas_call(
        _ffn_body,
        grid=(GRID,),
        in_specs=[_blk(ROWS, DIM), _blk(ROWS, DIM),
                  _full((DIM, DIM)), _full((1, DIM)),
                  _full((1, DIM)), _full((1, DIM)),
                  _full((DIM, 4 * DIM)), _full((1, 4 * DIM)),
                  _full((4 * DIM, DIM)), _full((1, DIM))],
        out_specs=_blk(ROWS, DIM),
        out_shape=jax.ShapeDtypeStruct((N, DIM), f32),
        compiler_params=_PAR,
    )(x_item, attn, Wout.astype(jnp.bfloat16), row(bout), row(ln2_s),
      row(ln2_b), Wf1.astype(jnp.bfloat16), row(bf1),
      Wf2.astype(jnp.bfloat16), row(bf2))
    return out
